# Initial kernel scaffold; baseline (speedup 1.0000x reference)
import jax
import jax.numpy as jnp
from jax.experimental import pallas as pl


def kernel(x, edge_index, batch, s1_Wl, s1_bl, s1_Wr, bn1_g, bn1_b, gW, gasrc, gadst, gbias, bn2_g, bn2_b, s2_Wl, s2_bl, s2_Wr, bn3_g, bn3_b, s3_Wl, s3_bl, s3_Wr, d1_W, d1_b, bnd1_g, bnd1_b, d2_W, d2_b, bnd2_g, bnd2_b, d3a_W, d3a_b, bnd3_g, bnd3_b, d3b_W, d3b_b):
    raise NotImplementedError("write your pallas kernel here")


# speedup vs baseline: 4.7496x; 4.7496x over previous
import functools

import jax
import jax.numpy as jnp
from jax import lax
from jax.experimental import pallas as pl
from jax.experimental.pallas import tpu as pltpu
from jax.experimental.pallas import tpu_sc as plsc

N = 10000
E = 160000
IN = 256
H = 512
OUT = 128
NS = 10
EPS = 1e-5

F = 128
EB = 64
NPAD = 10240
EPAD = 163840
NSUB = 16
NCORE = 2
ROWS_SUB = NPAD // NSUB
DUMMY_ROW = NPAD - 1

_MESH = plsc.VectorSubcoreMesh(core_axis_name="c", subcore_axis_name="s")
_SC_PARAMS = pltpu.CompilerParams(use_tc_tiling_on_sc=False,
                                  needs_layout_passes=False)


def _zero_vmem2d(ref, nrow, ncol):
    z16 = jnp.zeros((16,), jnp.float32)
    for i in range(nrow):
        for k in range(ncol // 16):
            ref[i, pl.ds(k * 16, 16)] = z16


def _zero_vmem1d(ref, n):
    z16 = jnp.zeros((16,), jnp.float32)
    for k in range(n // 16):
        ref[pl.ds(k * 16, 16)] = z16


def _splat_i32(v):
    return jnp.zeros((16,), jnp.int32) + v


def _sc_segsum(tables, src2d, dst2d, attn=None, with_deg=False):
    C = len(tables)
    weighted = attn is not None
    nblk = EPAD // NSUB // EB

    out_type = [jax.ShapeDtypeStruct((NPAD, F), jnp.float32) for _ in range(C)]
    if with_deg or weighted:
        out_type.append(jax.ShapeDtypeStruct((NPAD,), jnp.float32))

    scratch = [
        pltpu.VMEM((nblk, EB), jnp.int32),
        pltpu.VMEM((nblk, EB), jnp.int32),
        pltpu.VMEM((EB, F), jnp.float32),
        pltpu.VMEM((EB, F), jnp.float32),
        pltpu.VMEM((16, F), jnp.float32),
        pltpu.VMEM_SHARED((NPAD, F), jnp.float32),
        pltpu.SemaphoreType.DMA,
        pltpu.SemaphoreType.DMA,
        pltpu.SemaphoreType.DMA,
        pltpu.SemaphoreType.DMA,
    ]
    if weighted:
        scratch.append(pltpu.VMEM((EB,), jnp.float32))
        scratch.append(pltpu.VMEM((EB,), jnp.float32))
        scratch.append(pltpu.VMEM((EB,), jnp.float32))
        scratch.append(pltpu.VMEM((16,), jnp.float32))
        scratch.append(pltpu.SemaphoreType.DMA)
        scratch.append(pltpu.SemaphoreType.DMA)
    if with_deg:
        scratch.append(pltpu.VMEM((EB,), jnp.float32))
    if with_deg or weighted:
        scratch.append(pltpu.VMEM((ROWS_SUB,), jnp.float32))
        scratch.append(pltpu.VMEM_SHARED((NPAD,), jnp.float32))

    def body(*refs):
        it = iter(refs)
        tbls = [next(it) for _ in range(C)]
        src_h = next(it)
        dst_h = next(it)
        if weighted:
            asp_h = next(it)
            adp_h = next(it)
            mv_h = next(it)
        outs = [next(it) for _ in range(C)]
        scal_h = next(it) if (with_deg or weighted) else None
        sidx = next(it)
        didx = next(it)
        rows = (next(it), next(it))
        zbuf = next(it)
        acc = next(it)
        semg = (next(it), next(it))
        sems = (next(it), next(it))
        if weighted:
            bas = next(it)
            bad = next(it)
            bex = next(it)
            mv = next(it)
            sga = next(it)
            sgb = next(it)
        if with_deg:
            onesb = next(it)
        if with_deg or weighted:
            z1 = next(it)
            accd = next(it)

        cid = lax.axis_index("c")
        sid = lax.axis_index("s")
        blk0 = sid * nblk

        pltpu.sync_copy(src_h.at[pl.ds(blk0, nblk)], sidx)
        pltpu.sync_copy(dst_h.at[pl.ds(blk0, nblk)], didx)
        _zero_vmem2d(zbuf, 16, F)
        if weighted:
            pltpu.sync_copy(mv_h, mv)
        if with_deg:
            o16 = jnp.zeros((16,), jnp.float32) + 1.0
            for k in range(EB // 16):
                onesb[pl.ds(k * 16, 16)] = o16
        if with_deg or weighted:
            _zero_vmem1d(z1, ROWS_SUB)

        for ci in range(C):
            owner = ci % NCORE
            tbl = tbls[ci]
            outp = outs[ci]
            do_scal = (with_deg or weighted) and ci == 0

            @pl.when(cid == owner)
            def _chunk(tbl=tbl, outp=outp, do_scal=do_scal):
                def zb(i, carry):
                    pltpu.sync_copy(
                        zbuf, acc.at[pl.ds(sid * ROWS_SUB + i * 16, 16)])
                    return carry
                lax.fori_loop(0, ROWS_SUB // 16, zb, 0)
                if do_scal:
                    pltpu.sync_copy(z1, accd.at[pl.ds(sid * ROWS_SUB, ROWS_SUB)])
                plsc.subcore_barrier()

                def wait_g(p):
                    pltpu.make_async_copy(
                        tbl.at[pl.ds(0, EB)], rows[p], semg[p]).wait()

                def wait_s(p):
                    pltpu.make_async_copy(
                        tbl.at[pl.ds(0, EB)], rows[p], sems[p]).wait()

                pltpu.async_copy(tbl.at[sidx.at[0]], rows[0], semg[0])
                pltpu.async_copy(tbl.at[sidx.at[1]], rows[1], semg[1])

                def step(i, carry):
                    for p in (0, 1):
                        b = i * 2 + p
                        wait_g(p)
                        if weighted:
                            ga = pltpu.async_copy(
                                asp_h.at[sidx.at[b]], bas, sga)
                            gb = pltpu.async_copy(
                                adp_h.at[didx.at[b]], bad, sgb)
                            ga.wait()
                            gb.wait()
                            mvv = mv[pl.ds(0, 16)]
                            for k in range(EB // 16):
                                ev = (bas[pl.ds(k * 16, 16)]
                                      + bad[pl.ds(k * 16, 16)])
                                ev = jnp.maximum(ev, 0.2 * ev)
                                bex[pl.ds(k * 16, 16)] = jnp.exp(ev - mvv)

                            def w16(jv, c2, p=p):
                                for jj in range(16):
                                    j = jv * 16 + jj
                                    wv = plsc.load_gather(
                                        bex, [_splat_i32(j)])
                                    for k in range(F // 16):
                                        v = rows[p][j, pl.ds(k * 16, 16)]
                                        rows[p][j, pl.ds(k * 16, 16)] = v * wv
                                return c2
                            lax.fori_loop(0, EB // 16, w16, 0)
                        pltpu.async_copy(
                            rows[p], acc.at[didx.at[b]], sems[p], add=True)
                        if do_scal:
                            src_scal = bex if weighted else onesb
                            pltpu.sync_copy(
                                src_scal, accd.at[didx.at[b]], add=True)

                        @pl.when(b + 2 < nblk)
                        def _adv(p=p, b=b):
                            wait_s(p)
                            pltpu.async_copy(
                                tbl.at[sidx.at[b + 2]], rows[p], semg[p])
                    return carry

                lax.fori_loop(0, nblk // 2, step, 0)
                wait_s(0)
                wait_s(1)
                plsc.subcore_barrier()
                pltpu.sync_copy(
                    acc.at[pl.ds(sid * ROWS_SUB, ROWS_SUB)],
                    outp.at[pl.ds(sid * ROWS_SUB, ROWS_SUB)])
                if do_scal:
                    pltpu.sync_copy(
                        accd.at[pl.ds(sid * ROWS_SUB, ROWS_SUB)],
                        scal_h.at[pl.ds(sid * ROWS_SUB, ROWS_SUB)])
                plsc.subcore_barrier()

    ins = list(tables) + [src2d, dst2d]
    if weighted:
        ins += list(attn)
    fn = pl.kernel(body, out_type=tuple(out_type), mesh=_MESH,
                   scratch_types=scratch, compiler_params=_SC_PARAMS)
    return fn(*ins)


_BN = 1000


def _full_spec(shape):
    nd = len(shape)
    return pl.BlockSpec(shape, lambda i: (0,) * nd)


def _row_spec(bn, ncol):
    return pl.BlockSpec((bn, ncol), lambda i: (i, 0))


def _sage_tc(agg, deg, x, W1, W2, b1, scale, shift, W3=None, relu_bn=True):
    Nr, K = x.shape
    M = W2.shape[1]
    grid = (Nr // _BN,)
    KA = agg.shape[1]

    in_specs = [_row_spec(_BN, KA), _row_spec(_BN, 1), _row_spec(_BN, K)]
    ins = [agg, deg, x]
    if W1 is not None:
        in_specs.append(_full_spec((KA, M)))
        ins.append(W1)
    in_specs += [_full_spec((K, M)), _full_spec((1, M))]
    ins += [W2, b1.reshape(1, M)]
    if relu_bn:
        in_specs += [_full_spec((1, M)), _full_spec((1, M))]
        ins += [scale.reshape(1, M), shift.reshape(1, M)]
    if W3 is not None:
        M3 = W3.shape[1]
        in_specs.append(_full_spec((M, M3)))
        ins.append(W3)
        out_shape = (jax.ShapeDtypeStruct((Nr, M), jnp.float32),
                     jax.ShapeDtypeStruct((Nr, M3), jnp.float32))
        out_specs = (_row_spec(_BN, M), _row_spec(_BN, M3))
    else:
        out_shape = jax.ShapeDtypeStruct((Nr, M), jnp.float32)
        out_specs = _row_spec(_BN, M)

    def body(*refs):
        it = iter(refs)
        aggr = next(it)
        degr = next(it)
        xr = next(it)
        w1r = next(it) if W1 is not None else None
        w2r = next(it)
        b1r = next(it)
        if relu_bn:
            scr = next(it)
            shr = next(it)
        w3r = next(it) if W3 is not None else None
        outr = next(it)
        yr = next(it) if W3 is not None else None

        mean = aggr[...] * (1.0 / jnp.maximum(degr[...], 1.0))
        if W1 is not None:
            t = jnp.dot(mean, w1r[...], preferred_element_type=jnp.float32)
        else:
            t = mean
        t = t + jnp.dot(xr[...], w2r[...], preferred_element_type=jnp.float32)
        t = t + b1r[...]
        if relu_bn:
            t = jnp.maximum(t, 0.0)
            t = t * scr[...] + shr[...]
        outr[...] = t
        if W3 is not None:
            yr[...] = jnp.dot(t, w3r[...], preferred_element_type=jnp.float32)

    return pl.pallas_call(body, grid=grid, in_specs=in_specs,
                          out_specs=out_specs, out_shape=out_shape)(*ins)


def _gat_pre(x1, gW, A2p):
    Nr = x1.shape[0]
    grid = (Nr // _BN,)

    def body(xr, wr, ar, hr, asadr):
        hb = jnp.dot(xr[...], wr[...], preferred_element_type=jnp.float32)
        hr[...] = hb
        asadr[...] = jnp.dot(hb, ar[...], preferred_element_type=jnp.float32)

    return pl.pallas_call(
        body, grid=grid,
        in_specs=[_row_spec(_BN, H), _full_spec((H, H)), _full_spec((H, 128))],
        out_specs=(_row_spec(_BN, H), _row_spec(_BN, 128)),
        out_shape=(jax.ShapeDtypeStruct((Nr, H), jnp.float32),
                   jax.ShapeDtypeStruct((Nr, 128), jnp.float32)),
    )(x1, gW, A2p)


def _gat_post(num, den, h, asad, Marr, gbias, scale, shift):
    Nr = h.shape[0]
    grid = (Nr // _BN,)

    def body(numr, denr, hr, asadr, mr, gbr, scr, shr, outr):
        a_s = asadr[...][:, 0:1]
        a_d = asadr[...][:, 1:2]
        e = a_s + a_d
        e = jnp.maximum(e, 0.2 * e)
        exs = jnp.exp(e - mr[...][:, 0:1])
        numt = numr[...] + exs * hr[...]
        dent = denr[...] + exs
        o = numt / jnp.maximum(dent, 1e-16) + gbr[...]
        o = jnp.maximum(o, 0.0)
        outr[...] = o * scr[...] + shr[...]

    return pl.pallas_call(
        body, grid=grid,
        in_specs=[_row_spec(_BN, H), _row_spec(_BN, 1), _row_spec(_BN, H),
                  _row_spec(_BN, 128), _full_spec((1, 128)),
                  _full_spec((1, H)), _full_spec((1, H)), _full_spec((1, H))],
        out_specs=_row_spec(_BN, H),
        out_shape=jax.ShapeDtypeStruct((Nr, H), jnp.float32),
    )(num, den, h, asad, Marr, gbias.reshape(1, H), scale.reshape(1, H),
      shift.reshape(1, H))


def _decoder(mean, log_std, nm, x1, x2, d1_W, d1_b, sc1, sh1,
             d2a, d2b, d2_b, sc2, sh2, d3aa, d3ab, d3a_b, sc3, sh3,
             d3b_W, d3b_b):
    Nr = mean.shape[0]
    grid = (Nr // _BN,)

    def body(mr, lr, nmr, x1r, x2r, w1r, b1r, s1r, h1r,
             w2ar, w2br, b2r, s2r, h2r, w3ar, w3br, b3r, s3r, h3r,
             w4r, b4r, outr):
        z = mr[...] + jnp.exp(lr[...]) * nmr[...]
        h1 = jnp.dot(z, w1r[...], preferred_element_type=jnp.float32) + b1r[...]
        h1 = h1 * s1r[...] + h1r[...]
        h2 = (jnp.dot(h1, w2ar[...], preferred_element_type=jnp.float32)
              + jnp.dot(x1r[...], w2br[...], preferred_element_type=jnp.float32)
              + b2r[...])
        h2 = h2 * s2r[...] + h2r[...]
        h3 = (jnp.dot(h2, w3ar[...], preferred_element_type=jnp.float32)
              + jnp.dot(x2r[...], w3br[...], preferred_element_type=jnp.float32)
              + b3r[...])
        h3 = h3 * s3r[...] + h3r[...]
        outr[...] = (jnp.dot(h3, w4r[...], preferred_element_type=jnp.float32)
                     + b4r[...])

    in_specs = [
        _row_spec(_BN, OUT), _row_spec(_BN, OUT), _row_spec(_BN, OUT),
        _row_spec(_BN, H), _row_spec(_BN, H),
        _full_spec((OUT, H)), _full_spec((1, H)), _full_spec((1, H)),
        _full_spec((1, H)),
        _full_spec((H, H)), _full_spec((H, H)), _full_spec((1, H)),
        _full_spec((1, H)), _full_spec((1, H)),
        _full_spec((H, H)), _full_spec((H, H)), _full_spec((1, H)),
        _full_spec((1, H)), _full_spec((1, H)),
        _full_spec((H, IN)), _full_spec((1, IN)),
    ]
    return pl.pallas_call(
        body, grid=grid, in_specs=in_specs,
        out_specs=_row_spec(_BN, IN),
        out_shape=jax.ShapeDtypeStruct((Nr, IN), jnp.float32),
    )(mean, log_std, nm, x1, x2,
      d1_W, d1_b.reshape(1, H), sc1.reshape(1, H), sh1.reshape(1, H),
      d2a, d2b, d2_b.reshape(1, H), sc2.reshape(1, H), sh2.reshape(1, H),
      d3aa, d3ab, d3a_b.reshape(1, H), sc3.reshape(1, H), sh3.reshape(1, H),
      d3b_W, d3b_b.reshape(1, IN))


def kernel(x, edge_index, batch, s1_Wl, s1_bl, s1_Wr, bn1_g, bn1_b, gW, gasrc,
           gadst, gbias, bn2_g, bn2_b, s2_Wl, s2_bl, s2_Wr, bn3_g, bn3_b,
           s3_Wl, s3_bl, s3_Wr, d1_W, d1_b, bnd1_g, bnd1_b, d2_W, d2_b,
           bnd2_g, bnd2_b, d3a_W, d3a_b, bnd3_g, bnd3_b, d3b_W, d3b_b):
    f32 = jnp.float32
    bn_k = 1.0 / jnp.sqrt(jnp.float32(1.0 + EPS))

    src = edge_index[0]
    dst = edge_index[1]
    pad_e = EPAD - E
    src_p = jnp.concatenate([src, jnp.zeros((pad_e,), jnp.int32)])
    dst_p = jnp.concatenate([dst, jnp.full((pad_e,), DUMMY_ROW, jnp.int32)])
    src2d = src_p.reshape(EPAD // EB, EB)
    dst2d = dst_p.reshape(EPAD // EB, EB)

    xc = [x[:, i * F:(i + 1) * F] for i in range(IN // F)]
    res = _sc_segsum(xc, src2d, dst2d, with_deg=True)
    agg1 = jnp.concatenate([r[:N] for r in res[:-1]], axis=1)
    deg = res[-1][:N][:, None]
    x1 = _sage_tc(agg1, deg, x, s1_Wl, s1_Wr, s1_bl,
                  bn1_g * bn_k, bn1_b, relu_bn=True)

    A2p = jnp.concatenate(
        [gasrc[:, None], gadst[:, None], jnp.zeros((H, 126), f32)], axis=1)
    h, asad = _gat_pre(x1, gW, A2p)
    a_s = asad[:, 0]
    a_d = asad[:, 1]
    Mg = jnp.max(a_s) + jnp.max(a_d)
    zpadn = jnp.zeros((NPAD - N,), f32)
    asp = jnp.concatenate([a_s, zpadn])
    adp = jnp.concatenate([a_d, zpadn])
    mvec = jnp.full((16,), Mg, f32)
    hc = [h[:, i * F:(i + 1) * F] for i in range(H // F)]
    res2 = _sc_segsum(hc, src2d, dst2d, attn=(asp, adp, mvec))
    num = jnp.concatenate([r[:N] for r in res2[:-1]], axis=1)
    den = res2[-1][:N][:, None]
    Marr = jnp.full((1, 128), Mg, f32)
    x2 = _gat_post(num, den, h, asad, Marr, gbias, bn2_g * bn_k, bn2_b)

    x2c = [x2[:, i * F:(i + 1) * F] for i in range(H // F)]
    agg2c = _sc_segsum(x2c, src2d, dst2d)
    agg2 = jnp.concatenate([r[:N] for r in agg2c], axis=1)
    x3, y3 = _sage_tc(agg2, deg, x2, s2_Wl, s2_Wr, s2_bl,
                      bn3_g * bn_k, bn3_b, W3=s3_Wl, relu_bn=True)

    y3c = [y3[:, i * F:(i + 1) * F] for i in range(2 * OUT // F)]
    aggyc = _sc_segsum(y3c, src2d, dst2d)
    aggy = jnp.concatenate([r[:N] for r in aggyc], axis=1)
    ms = _sage_tc(aggy, deg, x3, None, s3_Wr, s3_bl, None, None,
                  relu_bn=False)
    mean = ms[:, :OUT]
    log_std = ms[:, OUT:]

    nm = jnp.mean(
        jax.random.normal(jax.random.key(42), (NS, N, OUT), f32), axis=0)
    recon = _decoder(
        mean, log_std, nm, x1, x2,
        d1_W, d1_b, bnd1_g * bn_k, bnd1_b,
        d2_W[:H], d2_W[H:], d2_b, bnd2_g * bn_k, bnd2_b,
        d3a_W[:H], d3a_W[H:], d3a_b, bnd3_g * bn_k, bnd3_b,
        d3b_W, d3b_b)
    return recon, mean, log_std


# speedup vs baseline: 5.1253x; 1.0791x over previous
import functools

import jax
import jax.numpy as jnp
from jax import lax
from jax.experimental import pallas as pl
from jax.experimental.pallas import tpu as pltpu
from jax.experimental.pallas import tpu_sc as plsc

N = 10000
E = 160000
IN = 256
H = 512
OUT = 128
NS = 10
EPS = 1e-5

F = 128
EB = 64
NPAD = 10240
EPAD = 163840
NSUB = 16
NCORE = 2
ROWS_SUB = NPAD // NSUB
DUMMY_ROW = NPAD - 1

_MESH = plsc.VectorSubcoreMesh(core_axis_name="c", subcore_axis_name="s")
_SC_PARAMS = pltpu.CompilerParams(use_tc_tiling_on_sc=False,
                                  needs_layout_passes=False)


def _zero_vmem2d(ref, nrow, ncol):
    z16 = jnp.zeros((16,), jnp.float32)
    for i in range(nrow):
        for k in range(ncol // 16):
            ref[i, pl.ds(k * 16, 16)] = z16


def _zero_vmem1d(ref, n):
    z16 = jnp.zeros((16,), jnp.float32)
    for k in range(n // 16):
        ref[pl.ds(k * 16, 16)] = z16


def _splat_i32(v):
    return jnp.zeros((16,), jnp.int32) + v


NBUF = 4
GLEAD = 2


def _sc_segsum(tables, packed2d, attn=None, with_deg=False):
    C = len(tables)
    weighted = attn is not None
    nblk = EPAD // NSUB // EB

    out_type = [jax.ShapeDtypeStruct((NPAD, F), jnp.float32) for _ in range(C)]
    if with_deg or weighted:
        out_type.append(jax.ShapeDtypeStruct((NPAD,), jnp.float32))

    scratch = [pltpu.VMEM((nblk, EB), jnp.int32)]
    scratch += [pltpu.VMEM((EB, F), jnp.float32) for _ in range(NBUF)]
    scratch += [pltpu.VMEM((EB,), jnp.int32) for _ in range(NBUF)]
    scratch += [pltpu.VMEM((EB,), jnp.int32) for _ in range(NBUF)]
    scratch.append(pltpu.VMEM((8, F), jnp.float32))
    scratch.append(pltpu.VMEM_SHARED((NPAD, F), jnp.float32))
    scratch += [pltpu.SemaphoreType.DMA for _ in range(2 * NBUF)]
    if weighted:
        scratch += [pltpu.VMEM((EB,), jnp.float32) for _ in range(NBUF)]
        scratch += [pltpu.VMEM((EB,), jnp.float32) for _ in range(NBUF)]
        scratch.append(pltpu.VMEM((EB,), jnp.float32))
        scratch.append(pltpu.VMEM((16,), jnp.float32))
        scratch += [pltpu.SemaphoreType.DMA for _ in range(2 * NBUF)]
    if with_deg:
        scratch.append(pltpu.VMEM((EB,), jnp.float32))
    if with_deg or weighted:
        scratch.append(pltpu.VMEM((ROWS_SUB,), jnp.float32))
        scratch.append(pltpu.VMEM_SHARED((NPAD,), jnp.float32))

    def body(*refs):
        it = iter(refs)
        tbls = [next(it) for _ in range(C)]
        pk_h = next(it)
        if weighted:
            asp_h = next(it)
            adp_h = next(it)
            mv_h = next(it)
        outs = [next(it) for _ in range(C)]
        scal_h = next(it) if (with_deg or weighted) else None
        pk = next(it)
        rows = [next(it) for _ in range(NBUF)]
        sidxb = [next(it) for _ in range(NBUF)]
        didxb = [next(it) for _ in range(NBUF)]
        zbuf = next(it)
        acc = next(it)
        semg = [next(it) for _ in range(NBUF)]
        sems = [next(it) for _ in range(NBUF)]
        if weighted:
            bas = [next(it) for _ in range(NBUF)]
            bad = [next(it) for _ in range(NBUF)]
            bex = next(it)
            mv = next(it)
            sga = [next(it) for _ in range(NBUF)]
            sgb = [next(it) for _ in range(NBUF)]
        if with_deg:
            onesb = next(it)
        if with_deg or weighted:
            z1 = next(it)
            accd = next(it)

        cid = lax.axis_index("c")
        sid = lax.axis_index("s")
        blk0 = sid * nblk

        pltpu.sync_copy(pk_h.at[pl.ds(blk0, nblk)], pk)
        _zero_vmem2d(zbuf, 8, F)
        if weighted:
            pltpu.sync_copy(mv_h, mv)
        if with_deg:
            o16 = jnp.zeros((16,), jnp.float32) + 1.0
            for k in range(EB // 16):
                onesb[pl.ds(k * 16, 16)] = o16
        if with_deg or weighted:
            _zero_vmem1d(z1, ROWS_SUB)

        def unpack(b, q):
            for k in range(EB // 16):
                p16 = pk[b, pl.ds(k * 16, 16)]
                sidxb[q][pl.ds(k * 16, 16)] = jnp.bitwise_and(p16, 16383)
                didxb[q][pl.ds(k * 16, 16)] = lax.shift_right_logical(p16, 14)

        for ci in range(C):
            owner = ci % NCORE
            tbl = tbls[ci]
            outp = outs[ci]
            do_scal = (with_deg or weighted) and ci == 0

            @pl.when(cid == owner)
            def _chunk(tbl=tbl, outp=outp, do_scal=do_scal):
                def zb(i, carry):
                    pltpu.sync_copy(
                        zbuf, acc.at[pl.ds(sid * ROWS_SUB + i * 8, 8)])
                    return carry
                lax.fori_loop(0, ROWS_SUB // 8, zb, 0)
                if do_scal:
                    pltpu.sync_copy(z1, accd.at[pl.ds(sid * ROWS_SUB, ROWS_SUB)])
                plsc.subcore_barrier()

                def issue_g(b, q):
                    unpack(b, q)
                    pltpu.async_copy(tbl.at[sidxb[q]], rows[q], semg[q])
                    if weighted:
                        pltpu.async_copy(asp_h.at[sidxb[q]], bas[q], sga[q])
                        pltpu.async_copy(adp_h.at[didxb[q]], bad[q], sgb[q])

                def wait_g(q):
                    pltpu.make_async_copy(
                        tbl.at[pl.ds(0, EB)], rows[q], semg[q]).wait()
                    if weighted:
                        pltpu.make_async_copy(
                            asp_h.at[pl.ds(0, EB)], bas[q], sga[q]).wait()
                        pltpu.make_async_copy(
                            adp_h.at[pl.ds(0, EB)], bad[q], sgb[q]).wait()

                def wait_s(q):
                    pltpu.make_async_copy(
                        tbl.at[pl.ds(0, EB)], rows[q], sems[q]).wait()

                for b in range(GLEAD):
                    issue_g(b, b % NBUF)

                def step(i, carry):
                    for p0 in range(NBUF):
                        j = i * NBUF + p0
                        p = p0
                        wait_g(p)
                        if weighted:
                            mvv = mv[pl.ds(0, 16)]
                            for k in range(EB // 16):
                                ev = (bas[p][pl.ds(k * 16, 16)]
                                      + bad[p][pl.ds(k * 16, 16)])
                                ev = jnp.maximum(ev, 0.2 * ev)
                                bex[pl.ds(k * 16, 16)] = jnp.exp(ev - mvv)

                            def w16(jv, c2, p=p):
                                for jj in range(16):
                                    jjj = jv * 16 + jj
                                    wv = plsc.load_gather(
                                        bex, [_splat_i32(jjj)])
                                    for k in range(F // 16):
                                        v = rows[p][jjj, pl.ds(k * 16, 16)]
                                        rows[p][jjj, pl.ds(k * 16, 16)] = v * wv
                                return c2
                            lax.fori_loop(0, EB // 16, w16, 0)
                        pltpu.async_copy(
                            rows[p], acc.at[didxb[p]], sems[p], add=True)
                        if do_scal:
                            src_scal = bex if weighted else onesb
                            pltpu.sync_copy(
                                src_scal, accd.at[didxb[p]], add=True)

                        q = (p0 + GLEAD) % NBUF

                        @pl.when(j + GLEAD < nblk)
                        def _adv(j=j, q=q):
                            @pl.when(j >= NBUF - GLEAD)
                            def _drain(q=q):
                                wait_s(q)
                            issue_g(j + GLEAD, q)
                    return carry

                lax.fori_loop(0, nblk // NBUF, step, 0)
                for q in range(NBUF):
                    wait_s(q)
                plsc.subcore_barrier()
                pltpu.sync_copy(
                    acc.at[pl.ds(sid * ROWS_SUB, ROWS_SUB)],
                    outp.at[pl.ds(sid * ROWS_SUB, ROWS_SUB)])
                if do_scal:
                    pltpu.sync_copy(
                        accd.at[pl.ds(sid * ROWS_SUB, ROWS_SUB)],
                        scal_h.at[pl.ds(sid * ROWS_SUB, ROWS_SUB)])
                plsc.subcore_barrier()

    ins = list(tables) + [packed2d]
    if weighted:
        ins += list(attn)
    fn = pl.kernel(body, out_type=tuple(out_type), mesh=_MESH,
                   scratch_types=scratch, compiler_params=_SC_PARAMS)
    return fn(*ins)


_BN = 1000


def _full_spec(shape):
    nd = len(shape)
    return pl.BlockSpec(shape, lambda i: (0,) * nd)


def _row_spec(bn, ncol):
    return pl.BlockSpec((bn, ncol), lambda i: (i, 0))


def _sage_tc(agg, deg, x, W1, W2, b1, scale, shift, W3=None, relu_bn=True):
    Nr, K = x.shape
    M = W2.shape[1]
    grid = (Nr // _BN,)
    KA = agg.shape[1]

    in_specs = [_row_spec(_BN, KA), _row_spec(_BN, 1), _row_spec(_BN, K)]
    ins = [agg, deg, x]
    if W1 is not None:
        in_specs.append(_full_spec((KA, M)))
        ins.append(W1)
    in_specs += [_full_spec((K, M)), _full_spec((1, M))]
    ins += [W2, b1.reshape(1, M)]
    if relu_bn:
        in_specs += [_full_spec((1, M)), _full_spec((1, M))]
        ins += [scale.reshape(1, M), shift.reshape(1, M)]
    if W3 is not None:
        M3 = W3.shape[1]
        in_specs.append(_full_spec((M, M3)))
        ins.append(W3)
        out_shape = (jax.ShapeDtypeStruct((Nr, M), jnp.float32),
                     jax.ShapeDtypeStruct((Nr, M3), jnp.float32))
        out_specs = (_row_spec(_BN, M), _row_spec(_BN, M3))
    else:
        out_shape = jax.ShapeDtypeStruct((Nr, M), jnp.float32)
        out_specs = _row_spec(_BN, M)

    def body(*refs):
        it = iter(refs)
        aggr = next(it)
        degr = next(it)
        xr = next(it)
        w1r = next(it) if W1 is not None else None
        w2r = next(it)
        b1r = next(it)
        if relu_bn:
            scr = next(it)
            shr = next(it)
        w3r = next(it) if W3 is not None else None
        outr = next(it)
        yr = next(it) if W3 is not None else None

        mean = aggr[...] * (1.0 / jnp.maximum(degr[...], 1.0))
        if W1 is not None:
            t = jnp.dot(mean, w1r[...], preferred_element_type=jnp.float32)
        else:
            t = mean
        t = t + jnp.dot(xr[...], w2r[...], preferred_element_type=jnp.float32)
        t = t + b1r[...]
        if relu_bn:
            t = jnp.maximum(t, 0.0)
            t = t * scr[...] + shr[...]
        outr[...] = t
        if W3 is not None:
            yr[...] = jnp.dot(t, w3r[...], preferred_element_type=jnp.float32)

    return pl.pallas_call(body, grid=grid, in_specs=in_specs,
                          out_specs=out_specs, out_shape=out_shape)(*ins)


def _gat_pre(x1, gW, A2p):
    Nr = x1.shape[0]
    grid = (Nr // _BN,)

    def body(xr, wr, ar, hr, asadr):
        hb = jnp.dot(xr[...], wr[...], preferred_element_type=jnp.float32)
        hr[...] = hb
        asadr[...] = jnp.dot(hb, ar[...], preferred_element_type=jnp.float32)

    return pl.pallas_call(
        body, grid=grid,
        in_specs=[_row_spec(_BN, H), _full_spec((H, H)), _full_spec((H, 128))],
        out_specs=(_row_spec(_BN, H), _row_spec(_BN, 128)),
        out_shape=(jax.ShapeDtypeStruct((Nr, H), jnp.float32),
                   jax.ShapeDtypeStruct((Nr, 128), jnp.float32)),
    )(x1, gW, A2p)


def _gat_post(num, den, h, asad, Marr, gbias, scale, shift):
    Nr = h.shape[0]
    grid = (Nr // _BN,)

    def body(numr, denr, hr, asadr, mr, gbr, scr, shr, outr):
        a_s = asadr[...][:, 0:1]
        a_d = asadr[...][:, 1:2]
        e = a_s + a_d
        e = jnp.maximum(e, 0.2 * e)
        exs = jnp.exp(e - mr[...][:, 0:1])
        numt = numr[...] + exs * hr[...]
        dent = denr[...] + exs
        o = numt / jnp.maximum(dent, 1e-16) + gbr[...]
        o = jnp.maximum(o, 0.0)
        outr[...] = o * scr[...] + shr[...]

    return pl.pallas_call(
        body, grid=grid,
        in_specs=[_row_spec(_BN, H), _row_spec(_BN, 1), _row_spec(_BN, H),
                  _row_spec(_BN, 128), _full_spec((1, 128)),
                  _full_spec((1, H)), _full_spec((1, H)), _full_spec((1, H))],
        out_specs=_row_spec(_BN, H),
        out_shape=jax.ShapeDtypeStruct((Nr, H), jnp.float32),
    )(num, den, h, asad, Marr, gbias.reshape(1, H), scale.reshape(1, H),
      shift.reshape(1, H))


def _decoder(mean, log_std, nm, x1, x2, d1_W, d1_b, sc1, sh1,
             d2a, d2b, d2_b, sc2, sh2, d3aa, d3ab, d3a_b, sc3, sh3,
             d3b_W, d3b_b):
    Nr = mean.shape[0]
    grid = (Nr // _BN,)

    def body(mr, lr, nmr, x1r, x2r, w1r, b1r, s1r, h1r,
             w2ar, w2br, b2r, s2r, h2r, w3ar, w3br, b3r, s3r, h3r,
             w4r, b4r, outr):
        z = mr[...] + jnp.exp(lr[...]) * nmr[...]
        h1 = jnp.dot(z, w1r[...], preferred_element_type=jnp.float32) + b1r[...]
        h1 = h1 * s1r[...] + h1r[...]
        h2 = (jnp.dot(h1, w2ar[...], preferred_element_type=jnp.float32)
              + jnp.dot(x1r[...], w2br[...], preferred_element_type=jnp.float32)
              + b2r[...])
        h2 = h2 * s2r[...] + h2r[...]
        h3 = (jnp.dot(h2, w3ar[...], preferred_element_type=jnp.float32)
              + jnp.dot(x2r[...], w3br[...], preferred_element_type=jnp.float32)
              + b3r[...])
        h3 = h3 * s3r[...] + h3r[...]
        outr[...] = (jnp.dot(h3, w4r[...], preferred_element_type=jnp.float32)
                     + b4r[...])

    in_specs = [
        _row_spec(_BN, OUT), _row_spec(_BN, OUT), _row_spec(_BN, OUT),
        _row_spec(_BN, H), _row_spec(_BN, H),
        _full_spec((OUT, H)), _full_spec((1, H)), _full_spec((1, H)),
        _full_spec((1, H)),
        _full_spec((H, H)), _full_spec((H, H)), _full_spec((1, H)),
        _full_spec((1, H)), _full_spec((1, H)),
        _full_spec((H, H)), _full_spec((H, H)), _full_spec((1, H)),
        _full_spec((1, H)), _full_spec((1, H)),
        _full_spec((H, IN)), _full_spec((1, IN)),
    ]
    return pl.pallas_call(
        body, grid=grid, in_specs=in_specs,
        out_specs=_row_spec(_BN, IN),
        out_shape=jax.ShapeDtypeStruct((Nr, IN), jnp.float32),
    )(mean, log_std, nm, x1, x2,
      d1_W, d1_b.reshape(1, H), sc1.reshape(1, H), sh1.reshape(1, H),
      d2a, d2b, d2_b.reshape(1, H), sc2.reshape(1, H), sh2.reshape(1, H),
      d3aa, d3ab, d3a_b.reshape(1, H), sc3.reshape(1, H), sh3.reshape(1, H),
      d3b_W, d3b_b.reshape(1, IN))


def kernel(x, edge_index, batch, s1_Wl, s1_bl, s1_Wr, bn1_g, bn1_b, gW, gasrc,
           gadst, gbias, bn2_g, bn2_b, s2_Wl, s2_bl, s2_Wr, bn3_g, bn3_b,
           s3_Wl, s3_bl, s3_Wr, d1_W, d1_b, bnd1_g, bnd1_b, d2_W, d2_b,
           bnd2_g, bnd2_b, d3a_W, d3a_b, bnd3_g, bnd3_b, d3b_W, d3b_b):
    f32 = jnp.float32
    bn_k = 1.0 / jnp.sqrt(jnp.float32(1.0 + EPS))

    src = edge_index[0]
    dst = edge_index[1]
    pad_e = EPAD - E
    src_p = jnp.concatenate([src, jnp.zeros((pad_e,), jnp.int32)])
    dst_p = jnp.concatenate([dst, jnp.full((pad_e,), DUMMY_ROW, jnp.int32)])
    packed2d = (src_p + (dst_p << 14)).reshape(EPAD // EB, EB)

    xc = [x[:, i * F:(i + 1) * F] for i in range(IN // F)]
    res = _sc_segsum(xc, packed2d, with_deg=True)
    agg1 = jnp.concatenate([r[:N] for r in res[:-1]], axis=1)
    deg = res[-1][:N][:, None]
    x1 = _sage_tc(agg1, deg, x, s1_Wl, s1_Wr, s1_bl,
                  bn1_g * bn_k, bn1_b, relu_bn=True)

    A2p = jnp.concatenate(
        [gasrc[:, None], gadst[:, None], jnp.zeros((H, 126), f32)], axis=1)
    h, asad = _gat_pre(x1, gW, A2p)
    a_s = asad[:, 0]
    a_d = asad[:, 1]
    Mg = jnp.max(a_s) + jnp.max(a_d)
    zpadn = jnp.zeros((NPAD - N,), f32)
    asp = jnp.concatenate([a_s, zpadn])
    adp = jnp.concatenate([a_d, zpadn])
    mvec = jnp.full((16,), Mg, f32)
    hc = [h[:, i * F:(i + 1) * F] for i in range(H // F)]
    res2 = _sc_segsum(hc, packed2d, attn=(asp, adp, mvec))
    num = jnp.concatenate([r[:N] for r in res2[:-1]], axis=1)
    den = res2[-1][:N][:, None]
    Marr = jnp.full((1, 128), Mg, f32)
    x2 = _gat_post(num, den, h, asad, Marr, gbias, bn2_g * bn_k, bn2_b)

    x2c = [x2[:, i * F:(i + 1) * F] for i in range(H // F)]
    agg2c = _sc_segsum(x2c, packed2d)
    agg2 = jnp.concatenate([r[:N] for r in agg2c], axis=1)
    x3, y3 = _sage_tc(agg2, deg, x2, s2_Wl, s2_Wr, s2_bl,
                      bn3_g * bn_k, bn3_b, W3=s3_Wl, relu_bn=True)

    y3c = [y3[:, i * F:(i + 1) * F] for i in range(2 * OUT // F)]
    aggyc = _sc_segsum(y3c, packed2d)
    aggy = jnp.concatenate([r[:N] for r in aggyc], axis=1)
    ms = _sage_tc(aggy, deg, x3, None, s3_Wr, s3_bl, None, None,
                  relu_bn=False)
    mean = ms[:, :OUT]
    log_std = ms[:, OUT:]

    nm = jnp.mean(
        jax.random.normal(jax.random.key(42), (NS, N, OUT), f32), axis=0)
    recon = _decoder(
        mean, log_std, nm, x1, x2,
        d1_W, d1_b, bnd1_g * bn_k, bnd1_b,
        d2_W[:H], d2_W[H:], d2_b, bnd2_g * bn_k, bnd2_b,
        d3a_W[:H], d3a_W[H:], d3a_b, bnd3_g * bn_k, bnd3_b,
        d3b_W, d3b_b)
    return recon, mean, log_std
